# baseline (speedup 1.0000x reference)
import functools

import jax
import jax.numpy as jnp
from jax import lax
from jax.experimental import pallas as pl
from jax.experimental.pallas import tpu as pltpu
from jax.experimental.pallas import tpu_sc as plsc

NC = 2
NS = 16
NW = NC * NS
EB = 128
CH = 16
TB0 = 80
TB1 = 80


def _scores_body(K128, nvalid, np_rows, d, x_ref, wp_ref, wrow_ref,
                 g_ref, kept_ref):
    X = x_ref[...]
    SB = jnp.dot(X, wp_ref[...], preferred_element_type=jnp.float32)
    w = wrow_ref[...]
    norm = jnp.sqrt(jnp.sum(w * w))
    ts = jnp.tanh(SB / norm)

    i = lax.bitcast_convert_type(ts, jnp.int32)
    key = i ^ (lax.shift_right_arithmetic(i, 31) & jnp.int32(0x7FFFFFFF))
    rows = lax.broadcasted_iota(jnp.int32, (np_rows, d), 0)
    valid = rows < nvalid

    def count(pred):
        return jnp.sum(pred.astype(jnp.int32))

    neg_inf = jnp.int32(-2**31)
    c0 = count(valid & (key >= jnp.int32(0)))
    T0 = jnp.where(c0 >= K128, jnp.int32(0), neg_inf)

    def sbody(it, T):
        cand = T + lax.shift_left(jnp.int32(1), 30 - it)
        c = count(valid & (key >= cand))
        return jnp.where(c >= K128, cand, T)

    T = lax.fori_loop(0, 31, sbody, T0)

    cgt = count(valid & (key > T))
    m128 = K128 - cgt
    tie = valid & (key == T)

    def tbody(it, T2):
        cand = T2 + lax.shift_left(jnp.int32(1), 13 - it)
        c = count(tie & (rows < cand))
        return jnp.where(c <= m128, cand, T2)

    T2 = lax.fori_loop(0, 14, tbody, jnp.int32(0))

    kept = valid & ((key > T) | (tie & (rows < T2)))
    g_ref[...] = jnp.where(kept, ts, 0.0)
    kept_ref[...] = kept.astype(jnp.float32)


def _mlp1_body(x_ref, a0_ref, a1_ref, g_ref, W1_ref, b1_ref, W2_ref, b2_ref,
               y_ref):
    h = x_ref[...] + a0_ref[...] + a1_ref[...]
    t = jnp.dot(h, W1_ref[...], preferred_element_type=jnp.float32) + b1_ref[...]
    t = jnp.maximum(t, 0.0)
    t = jnp.dot(t, W2_ref[...], preferred_element_type=jnp.float32) + b2_ref[...]
    t = jnp.maximum(t, 0.0)
    y_ref[...] = t * g_ref[...]


def _mlp2_body(nsteps, inv_k, y_ref, a0_ref, a1_ref, kept_ref,
               W3_ref, b3_ref, W4_ref, b4_ref, o_ref):
    kp = kept_ref[...]
    z = (y_ref[...] + a0_ref[...] + a1_ref[...]) * kp
    t = jnp.dot(z, W3_ref[...], preferred_element_type=jnp.float32) + b3_ref[...]
    t = jnp.maximum(t, 0.0)
    t = jnp.dot(t, W4_ref[...], preferred_element_type=jnp.float32) + b4_ref[...]
    t = jnp.maximum(t, 0.0) * kp

    @pl.when(pl.program_id(0) == 0)
    def _():
        o_ref[...] = jnp.zeros_like(o_ref)

    o_ref[...] += jnp.sum(t, axis=0, keepdims=True)

    @pl.when(pl.program_id(0) == nsteps - 1)
    def _():
        o_ref[...] *= inv_k


def _make_segsum(np_rows, d):
    mesh = plsc.VectorSubcoreMesh(core_axis_name="c", subcore_axis_name="s",
                                  num_cores=NC, num_subcores=NS)
    rows_per_tile = np_rows // NS
    tb_total = NS * (TB0 + TB1)

    @functools.partial(
        pl.kernel,
        out_type=jax.ShapeDtypeStruct((NC, np_rows, d), jnp.float32),
        mesh=mesh,
        scratch_types=[
            pltpu.VMEM((CH, EB), jnp.int32),
            pltpu.VMEM((CH, EB), jnp.int32),
            pltpu.VMEM((EB, d), jnp.float32),
            pltpu.VMEM((EB, d), jnp.float32),
            pltpu.VMEM_SHARED((np_rows, d), jnp.float32),
            pltpu.SemaphoreType.DMA,
            pltpu.SemaphoreType.DMA,
        ],
    )
    def segsum(tables, srcs, dsts, zrows, out,
               src_v, dst_v, buf_a, buf_b, acc, sem_a, sem_b):
        c = lax.axis_index("c")
        s = lax.axis_index("s")
        table = tables.at[c]
        r0 = s * rows_per_tile
        base = jnp.where(c == 0, s * TB0, NS * TB0 + s * TB1)
        n_chunks = jnp.where(c == 0, TB0 // CH, TB1 // CH)

        pltpu.sync_copy(zrows.at[pl.ds(r0, rows_per_tile)],
                        acc.at[pl.ds(r0, rows_per_tile)])
        plsc.subcore_barrier()

        @pl.loop(0, n_chunks)
        def _(ch):
            b0 = base + ch * CH
            pltpu.sync_copy(srcs.at[pl.ds(b0, CH)], src_v)
            pltpu.sync_copy(dsts.at[pl.ds(b0, CH)], dst_v)
            pltpu.async_copy(table.at[src_v.at[0]], buf_a, sem_a)

            @pl.loop(0, CH, step=2)
            def _(b):
                pltpu.async_copy(table.at[src_v.at[b + 1]], buf_b, sem_b)
                pltpu.make_async_copy(table.at[src_v.at[b]], buf_a,
                                      sem_a).wait()
                pltpu.sync_copy(buf_a, acc.at[dst_v.at[b]], add=True)

                @pl.when(b + 2 < CH)
                def _():
                    pltpu.async_copy(table.at[src_v.at[b + 2]], buf_a, sem_a)

                pltpu.make_async_copy(table.at[src_v.at[b + 1]], buf_b,
                                      sem_b).wait()
                pltpu.sync_copy(buf_b, acc.at[dst_v.at[b + 1]], add=True)

        plsc.subcore_barrier()
        pltpu.sync_copy(acc.at[pl.ds(r0, rows_per_tile)],
                        out.at[c, pl.ds(r0, rows_per_tile)])

    return segsum


def kernel(x, edge_index, batch, W1, b1, W2, b2, w_pool, W3, b3, W4, b4):
    n, d = x.shape
    e = edge_index.shape[1]
    k = (n + 1) // 2

    np_rows = -(-(n + 1) // 128) * 128
    rb = np_rows // 8
    tb_total = NS * (TB0 + TB1)
    ep = tb_total * EB
    assert ep >= e and TB0 % CH == 0 and TB1 % CH == 0

    x_pad = jnp.zeros((np_rows, d), jnp.float32).at[:n].set(x)
    srcp = jnp.full((ep,), n, jnp.int32).at[:e].set(edge_index[0])
    dstp = jnp.full((ep,), n, jnp.int32).at[:e].set(edge_index[1])
    srcp = srcp.reshape(tb_total, EB)
    dstp = dstp.reshape(tb_total, EB)
    zrows = jnp.zeros((np_rows, d), jnp.float32)
    WP = jnp.broadcast_to(w_pool[:, None], (d, d))
    wrow = w_pool.reshape(1, d)
    b1r, b2r, b3r, b4r = (bb.reshape(1, d) for bb in (b1, b2, b3, b4))

    f32 = jnp.float32
    g, kept = pl.pallas_call(
        functools.partial(_scores_body, k * d, n, np_rows, d),
        out_shape=[jax.ShapeDtypeStruct((np_rows, d), f32),
                   jax.ShapeDtypeStruct((np_rows, d), f32)],
    )(x_pad, WP, wrow)

    segsum = _make_segsum(np_rows, d)

    def two_parts(res):
        return (res[0], res[1]) if NC == 2 else (res[0], zrows)

    def seg2(table):
        t2 = jnp.broadcast_to(table, (NC, np_rows, d))
        return two_parts(segsum(t2, srcp, dstp, zrows))

    a = seg2(x_pad)

    nsteps = np_rows // rb
    row_spec = pl.BlockSpec((rb, d), lambda i: (i, 0))
    mat_spec = pl.BlockSpec((d, d), lambda i: (0, 0))
    vec_spec = pl.BlockSpec((1, d), lambda i: (0, 0))

    y = pl.pallas_call(
        _mlp1_body,
        grid=(nsteps,),
        in_specs=[row_spec, row_spec, row_spec, row_spec,
                  mat_spec, vec_spec, mat_spec, vec_spec],
        out_specs=row_spec,
        out_shape=jax.ShapeDtypeStruct((np_rows, d), f32),
    )(x_pad, a[0], a[1], g, W1, b1r, W2, b2r)

    a2 = seg2(y)

    out = pl.pallas_call(
        functools.partial(_mlp2_body, nsteps, 1.0 / float(k)),
        grid=(nsteps,),
        in_specs=[row_spec, row_spec, row_spec, row_spec,
                  mat_spec, vec_spec, mat_spec, vec_spec],
        out_specs=vec_spec,
        out_shape=jax.ShapeDtypeStruct((1, d), f32),
    )(y, a2[0], a2[1], kept, W3, b3r, W4, b4r)

    return out


# speedup vs baseline: 1.0701x; 1.0701x over previous
import functools

import jax
import jax.numpy as jnp
from jax import lax
from jax.experimental import pallas as pl
from jax.experimental.pallas import tpu as pltpu
from jax.experimental.pallas import tpu_sc as plsc

NC = 2
NS = 16
NW = NC * NS
EB = 128
CH = 16
TB0 = 128
TB1 = 32


def _scores_body(K128, nvalid, np_rows, d, x_ref, wp_ref, wrow_ref,
                 g_ref, kept_ref):
    X = x_ref[...]
    SB = jnp.dot(X, wp_ref[...], preferred_element_type=jnp.float32)
    w = wrow_ref[...]
    norm = jnp.sqrt(jnp.sum(w * w))
    ts = jnp.tanh(SB / norm)

    i = lax.bitcast_convert_type(ts, jnp.int32)
    key = i ^ (lax.shift_right_arithmetic(i, 31) & jnp.int32(0x7FFFFFFF))
    rows = lax.broadcasted_iota(jnp.int32, (np_rows, d), 0)
    valid = rows < nvalid

    def count(pred):
        return jnp.sum(pred.astype(jnp.int32))

    neg_inf = jnp.int32(-2**31)
    c0 = count(valid & (key >= jnp.int32(0)))
    T0 = jnp.where(c0 >= K128, jnp.int32(0), neg_inf)

    def sbody(it, T):
        cand = T + lax.shift_left(jnp.int32(1), 30 - it)
        c = count(valid & (key >= cand))
        return jnp.where(c >= K128, cand, T)

    T = lax.fori_loop(0, 31, sbody, T0)

    cgt = count(valid & (key > T))
    m128 = K128 - cgt
    tie = valid & (key == T)

    def tbody(it, T2):
        cand = T2 + lax.shift_left(jnp.int32(1), 13 - it)
        c = count(tie & (rows < cand))
        return jnp.where(c <= m128, cand, T2)

    T2 = lax.fori_loop(0, 14, tbody, jnp.int32(0))

    kept = valid & ((key > T) | (tie & (rows < T2)))
    g_ref[...] = jnp.where(kept, ts, 0.0)
    kept_ref[...] = kept.astype(jnp.float32)


def _mlp1_body(x_ref, a0_ref, a1_ref, g_ref, W1_ref, b1_ref, W2_ref, b2_ref,
               y_ref):
    h = x_ref[...] + a0_ref[...] + a1_ref[...]
    t = jnp.dot(h, W1_ref[...], preferred_element_type=jnp.float32) + b1_ref[...]
    t = jnp.maximum(t, 0.0)
    t = jnp.dot(t, W2_ref[...], preferred_element_type=jnp.float32) + b2_ref[...]
    t = jnp.maximum(t, 0.0)
    y_ref[...] = t * g_ref[...]


def _mlp2_body(nsteps, inv_k, y_ref, a0_ref, a1_ref, kept_ref,
               W3_ref, b3_ref, W4_ref, b4_ref, o_ref):
    kp = kept_ref[...]
    z = (y_ref[...] + a0_ref[...] + a1_ref[...]) * kp
    t = jnp.dot(z, W3_ref[...], preferred_element_type=jnp.float32) + b3_ref[...]
    t = jnp.maximum(t, 0.0)
    t = jnp.dot(t, W4_ref[...], preferred_element_type=jnp.float32) + b4_ref[...]
    t = jnp.maximum(t, 0.0) * kp

    @pl.when(pl.program_id(0) == 0)
    def _():
        o_ref[...] = jnp.zeros_like(o_ref)

    o_ref[...] += jnp.sum(t, axis=0, keepdims=True)

    @pl.when(pl.program_id(0) == nsteps - 1)
    def _():
        o_ref[...] *= inv_k


def _make_segsum(np_rows, d):
    mesh = plsc.VectorSubcoreMesh(core_axis_name="c", subcore_axis_name="s",
                                  num_cores=NC, num_subcores=NS)
    rows_per_tile = np_rows // NS
    tb_total = NS * (TB0 + TB1)

    @functools.partial(
        pl.kernel,
        out_type=jax.ShapeDtypeStruct((NC, np_rows, d), jnp.float32),
        mesh=mesh,
        scratch_types=[
            pltpu.VMEM((CH, EB), jnp.int32),
            pltpu.VMEM((CH, EB), jnp.int32),
            pltpu.VMEM((EB, d), jnp.float32),
            pltpu.VMEM((EB, d), jnp.float32),
            pltpu.VMEM_SHARED((np_rows, d), jnp.float32),
            pltpu.SemaphoreType.DMA,
            pltpu.SemaphoreType.DMA,
        ],
    )
    def segsum(tables, srcs, dsts, zrows, out,
               src_v, dst_v, buf_a, buf_b, acc, sem_a, sem_b):
        c = lax.axis_index("c")
        s = lax.axis_index("s")
        table = tables.at[c]
        r0 = s * rows_per_tile
        base = jnp.where(c == 0, s * TB0, NS * TB0 + s * TB1)
        n_chunks = jnp.where(c == 0, TB0 // CH, TB1 // CH)

        pltpu.sync_copy(zrows.at[pl.ds(r0, rows_per_tile)],
                        acc.at[pl.ds(r0, rows_per_tile)])
        plsc.subcore_barrier()

        @pl.loop(0, n_chunks)
        def _(ch):
            b0 = base + ch * CH
            pltpu.sync_copy(srcs.at[pl.ds(b0, CH)], src_v)
            pltpu.sync_copy(dsts.at[pl.ds(b0, CH)], dst_v)
            pltpu.async_copy(table.at[src_v.at[0]], buf_a, sem_a)

            @pl.loop(0, CH, step=2)
            def _(b):
                pltpu.async_copy(table.at[src_v.at[b + 1]], buf_b, sem_b)
                pltpu.make_async_copy(table.at[src_v.at[b]], buf_a,
                                      sem_a).wait()
                pltpu.sync_copy(buf_a, acc.at[dst_v.at[b]], add=True)

                @pl.when(b + 2 < CH)
                def _():
                    pltpu.async_copy(table.at[src_v.at[b + 2]], buf_a, sem_a)

                pltpu.make_async_copy(table.at[src_v.at[b + 1]], buf_b,
                                      sem_b).wait()
                pltpu.sync_copy(buf_b, acc.at[dst_v.at[b + 1]], add=True)

        plsc.subcore_barrier()
        pltpu.sync_copy(acc.at[pl.ds(r0, rows_per_tile)],
                        out.at[c, pl.ds(r0, rows_per_tile)])

    return segsum


def kernel(x, edge_index, batch, W1, b1, W2, b2, w_pool, W3, b3, W4, b4):
    n, d = x.shape
    e = edge_index.shape[1]
    k = (n + 1) // 2

    np_rows = -(-(n + 1) // 128) * 128
    rb = np_rows // 8
    tb_total = NS * (TB0 + TB1)
    ep = tb_total * EB
    assert ep >= e and TB0 % CH == 0 and TB1 % CH == 0

    x_pad = jnp.zeros((np_rows, d), jnp.float32).at[:n].set(x)
    srcp = jnp.full((ep,), n, jnp.int32).at[:e].set(edge_index[0])
    dstp = jnp.full((ep,), n, jnp.int32).at[:e].set(edge_index[1])
    srcp = srcp.reshape(tb_total, EB)
    dstp = dstp.reshape(tb_total, EB)
    zrows = jnp.zeros((np_rows, d), jnp.float32)
    WP = jnp.broadcast_to(w_pool[:, None], (d, d))
    wrow = w_pool.reshape(1, d)
    b1r, b2r, b3r, b4r = (bb.reshape(1, d) for bb in (b1, b2, b3, b4))

    f32 = jnp.float32
    g, kept = pl.pallas_call(
        functools.partial(_scores_body, k * d, n, np_rows, d),
        out_shape=[jax.ShapeDtypeStruct((np_rows, d), f32),
                   jax.ShapeDtypeStruct((np_rows, d), f32)],
    )(x_pad, WP, wrow)

    segsum = _make_segsum(np_rows, d)

    def two_parts(res):
        return (res[0], res[1]) if NC == 2 else (res[0], zrows)

    def seg2(table):
        t2 = jnp.broadcast_to(table, (NC, np_rows, d))
        return two_parts(segsum(t2, srcp, dstp, zrows))

    a = seg2(x_pad)

    nsteps = np_rows // rb
    row_spec = pl.BlockSpec((rb, d), lambda i: (i, 0))
    mat_spec = pl.BlockSpec((d, d), lambda i: (0, 0))
    vec_spec = pl.BlockSpec((1, d), lambda i: (0, 0))

    y = pl.pallas_call(
        _mlp1_body,
        grid=(nsteps,),
        in_specs=[row_spec, row_spec, row_spec, row_spec,
                  mat_spec, vec_spec, mat_spec, vec_spec],
        out_specs=row_spec,
        out_shape=jax.ShapeDtypeStruct((np_rows, d), f32),
    )(x_pad, a[0], a[1], g, W1, b1r, W2, b2r)

    a2 = seg2(y)

    out = pl.pallas_call(
        functools.partial(_mlp2_body, nsteps, 1.0 / float(k)),
        grid=(nsteps,),
        in_specs=[row_spec, row_spec, row_spec, row_spec,
                  mat_spec, vec_spec, mat_spec, vec_spec],
        out_specs=vec_spec,
        out_shape=jax.ShapeDtypeStruct((1, d), f32),
    )(y, a2[0], a2[1], kept, W3, b3r, W4, b4r)

    return out


# speedup vs baseline: 1.1296x; 1.0556x over previous
import functools

import jax
import jax.numpy as jnp
from jax import lax
from jax.experimental import pallas as pl
from jax.experimental.pallas import tpu as pltpu
from jax.experimental.pallas import tpu_sc as plsc

NC = 2
NS = 16
NW = NC * NS
EB = 128
CH = 16
TB0 = 144
TB1 = 16


def _scores_body(K128, nvalid, np_rows, d, x_ref, wp_ref, wrow_ref,
                 g_ref, kept_ref):
    X = x_ref[...]
    SB = jnp.dot(X, wp_ref[...], preferred_element_type=jnp.float32)
    w = wrow_ref[...]
    norm = jnp.sqrt(jnp.sum(w * w))
    ts = jnp.tanh(SB / norm)

    i = lax.bitcast_convert_type(ts, jnp.int32)
    key = i ^ (lax.shift_right_arithmetic(i, 31) & jnp.int32(0x7FFFFFFF))
    rows = lax.broadcasted_iota(jnp.int32, (np_rows, d), 0)
    valid = rows < nvalid

    def count(pred):
        return jnp.sum(pred.astype(jnp.int32))

    neg_inf = jnp.int32(-2**31)
    c0 = count(valid & (key >= jnp.int32(0)))
    T0 = jnp.where(c0 >= K128, jnp.int32(0), neg_inf)

    def sbody(it, T):
        cand = T + lax.shift_left(jnp.int32(1), 30 - it)
        c = count(valid & (key >= cand))
        return jnp.where(c >= K128, cand, T)

    T = lax.fori_loop(0, 31, sbody, T0)

    cgt = count(valid & (key > T))
    m128 = K128 - cgt
    tie = valid & (key == T)

    def tbody(it, T2):
        cand = T2 + lax.shift_left(jnp.int32(1), 13 - it)
        c = count(tie & (rows < cand))
        return jnp.where(c <= m128, cand, T2)

    T2 = lax.fori_loop(0, 14, tbody, jnp.int32(0))

    kept = valid & ((key > T) | (tie & (rows < T2)))
    g_ref[...] = jnp.where(kept, ts, 0.0)
    kept_ref[...] = kept.astype(jnp.float32)


def _mlp1_body(x_ref, a0_ref, a1_ref, g_ref, W1_ref, b1_ref, W2_ref, b2_ref,
               y_ref):
    h = x_ref[...] + a0_ref[...] + a1_ref[...]
    t = jnp.dot(h, W1_ref[...], preferred_element_type=jnp.float32) + b1_ref[...]
    t = jnp.maximum(t, 0.0)
    t = jnp.dot(t, W2_ref[...], preferred_element_type=jnp.float32) + b2_ref[...]
    t = jnp.maximum(t, 0.0)
    y_ref[...] = t * g_ref[...]


def _mlp2_body(nsteps, inv_k, y_ref, a0_ref, a1_ref, kept_ref,
               W3_ref, b3_ref, W4_ref, b4_ref, o_ref):
    kp = kept_ref[...]
    z = (y_ref[...] + a0_ref[...] + a1_ref[...]) * kp
    t = jnp.dot(z, W3_ref[...], preferred_element_type=jnp.float32) + b3_ref[...]
    t = jnp.maximum(t, 0.0)
    t = jnp.dot(t, W4_ref[...], preferred_element_type=jnp.float32) + b4_ref[...]
    t = jnp.maximum(t, 0.0) * kp

    @pl.when(pl.program_id(0) == 0)
    def _():
        o_ref[...] = jnp.zeros_like(o_ref)

    o_ref[...] += jnp.sum(t, axis=0, keepdims=True)

    @pl.when(pl.program_id(0) == nsteps - 1)
    def _():
        o_ref[...] *= inv_k


def _make_segsum(np_rows, d):
    mesh = plsc.VectorSubcoreMesh(core_axis_name="c", subcore_axis_name="s",
                                  num_cores=NC, num_subcores=NS)
    rows_per_tile = np_rows // NS
    tb_total = NS * (TB0 + TB1)

    @functools.partial(
        pl.kernel,
        out_type=jax.ShapeDtypeStruct((NC, np_rows, d), jnp.float32),
        mesh=mesh,
        scratch_types=[
            pltpu.VMEM((CH, EB), jnp.int32),
            pltpu.VMEM((CH, EB), jnp.int32),
            pltpu.VMEM((EB, d), jnp.float32),
            pltpu.VMEM((EB, d), jnp.float32),
            pltpu.VMEM_SHARED((np_rows, d), jnp.float32),
            pltpu.SemaphoreType.DMA,
            pltpu.SemaphoreType.DMA,
        ],
    )
    def segsum(tables, srcs, dsts, zrows, out,
               src_v, dst_v, buf_a, buf_b, acc, sem_a, sem_b):
        c = lax.axis_index("c")
        s = lax.axis_index("s")
        table = tables.at[c]
        r0 = s * rows_per_tile
        base = jnp.where(c == 0, s * TB0, NS * TB0 + s * TB1)
        n_chunks = jnp.where(c == 0, TB0 // CH, TB1 // CH)

        pltpu.sync_copy(zrows.at[pl.ds(r0, rows_per_tile)],
                        acc.at[pl.ds(r0, rows_per_tile)])
        plsc.subcore_barrier()

        @pl.loop(0, n_chunks)
        def _(ch):
            b0 = base + ch * CH
            pltpu.sync_copy(srcs.at[pl.ds(b0, CH)], src_v)
            pltpu.sync_copy(dsts.at[pl.ds(b0, CH)], dst_v)
            pltpu.async_copy(table.at[src_v.at[0]], buf_a, sem_a)

            @pl.loop(0, CH, step=2)
            def _(b):
                pltpu.async_copy(table.at[src_v.at[b + 1]], buf_b, sem_b)
                pltpu.make_async_copy(table.at[src_v.at[b]], buf_a,
                                      sem_a).wait()
                pltpu.sync_copy(buf_a, acc.at[dst_v.at[b]], add=True)

                @pl.when(b + 2 < CH)
                def _():
                    pltpu.async_copy(table.at[src_v.at[b + 2]], buf_a, sem_a)

                pltpu.make_async_copy(table.at[src_v.at[b + 1]], buf_b,
                                      sem_b).wait()
                pltpu.sync_copy(buf_b, acc.at[dst_v.at[b + 1]], add=True)

        plsc.subcore_barrier()
        pltpu.sync_copy(acc.at[pl.ds(r0, rows_per_tile)],
                        out.at[c, pl.ds(r0, rows_per_tile)])

    return segsum


def kernel(x, edge_index, batch, W1, b1, W2, b2, w_pool, W3, b3, W4, b4):
    n, d = x.shape
    e = edge_index.shape[1]
    k = (n + 1) // 2

    np_rows = -(-(n + 1) // 128) * 128
    rb = np_rows // 8
    tb_total = NS * (TB0 + TB1)
    ep = tb_total * EB
    assert ep >= e and TB0 % CH == 0 and TB1 % CH == 0

    x_pad = jnp.zeros((np_rows, d), jnp.float32).at[:n].set(x)
    srcp = jnp.full((ep,), n, jnp.int32).at[:e].set(edge_index[0])
    dstp = jnp.full((ep,), n, jnp.int32).at[:e].set(edge_index[1])
    srcp = srcp.reshape(tb_total, EB)
    dstp = dstp.reshape(tb_total, EB)
    zrows = jnp.zeros((np_rows, d), jnp.float32)
    WP = jnp.broadcast_to(w_pool[:, None], (d, d))
    wrow = w_pool.reshape(1, d)
    b1r, b2r, b3r, b4r = (bb.reshape(1, d) for bb in (b1, b2, b3, b4))

    f32 = jnp.float32
    g, kept = pl.pallas_call(
        functools.partial(_scores_body, k * d, n, np_rows, d),
        out_shape=[jax.ShapeDtypeStruct((np_rows, d), f32),
                   jax.ShapeDtypeStruct((np_rows, d), f32)],
    )(x_pad, WP, wrow)

    segsum = _make_segsum(np_rows, d)

    def two_parts(res):
        return (res[0], res[1]) if NC == 2 else (res[0], zrows)

    def seg2(table):
        t2 = jnp.broadcast_to(table, (NC, np_rows, d))
        return two_parts(segsum(t2, srcp, dstp, zrows))

    a = seg2(x_pad)

    nsteps = np_rows // rb
    row_spec = pl.BlockSpec((rb, d), lambda i: (i, 0))
    mat_spec = pl.BlockSpec((d, d), lambda i: (0, 0))
    vec_spec = pl.BlockSpec((1, d), lambda i: (0, 0))

    y = pl.pallas_call(
        _mlp1_body,
        grid=(nsteps,),
        in_specs=[row_spec, row_spec, row_spec, row_spec,
                  mat_spec, vec_spec, mat_spec, vec_spec],
        out_specs=row_spec,
        out_shape=jax.ShapeDtypeStruct((np_rows, d), f32),
    )(x_pad, a[0], a[1], g, W1, b1r, W2, b2r)

    a2 = seg2(y)

    out = pl.pallas_call(
        functools.partial(_mlp2_body, nsteps, 1.0 / float(k)),
        grid=(nsteps,),
        in_specs=[row_spec, row_spec, row_spec, row_spec,
                  mat_spec, vec_spec, mat_spec, vec_spec],
        out_specs=vec_spec,
        out_shape=jax.ShapeDtypeStruct((1, d), f32),
    )(y, a2[0], a2[1], kept, W3, b3r, W4, b4r)

    return out
